# baseline (speedup 1.0000x reference)
import functools

import jax
import jax.numpy as jnp
import numpy as np
from jax.experimental import pallas as pl
from jax.experimental.pallas import tpu as pltpu

B, T, D, K = 4096, 64, 256, 16
BB = 128

_Q_LO = 22
_Q_FRAC = np.float32(0.35 * (T - 1) - _Q_LO)
_NEG = np.float32(np.finfo(np.float32).min)


def _main_body(ao_ref, no_ref, f_ref, proj_ref, topk_ref, nmax_ref):
    f = f_ref[...].reshape(BB * T, D)
    scores = jax.lax.dot_general(
        f, proj_ref[...], (((1,), (0,)), ((), ())),
        preferred_element_type=jnp.float32)
    ids = jnp.argmax(scores, axis=-1).astype(jnp.float32).reshape(BB, T)

    ao = ao_ref[...]
    le = (ao[:, :, None] <= ao[:, None, :]).astype(jnp.float32)
    cnt = jnp.sum(le, axis=1)
    v_lo = jnp.min(jnp.where(cnt >= _Q_LO + 1, ao, jnp.inf), axis=1)
    v_hi = jnp.min(jnp.where(cnt >= _Q_LO + 2, ao, jnp.inf), axis=1)
    th = v_lo + _Q_FRAC * (v_hi - v_lo)

    masked_ao = jnp.where(ao >= th[:, None], ao, _NEG)
    vsum = jnp.zeros((BB,), jnp.float32)
    ncl = jnp.zeros((BB,), jnp.float32)
    for k in range(K):
        cmax_k = jnp.max(jnp.where(ids == float(k), masked_ao, _NEG),
                         axis=1)
        present = cmax_k > _NEG
        vsum = vsum + jnp.where(present, cmax_k, 0.0)
        ncl = ncl + present.astype(jnp.float32)
    topk_ref[...] = vsum / jnp.maximum(ncl, 1.0)
    nmax_ref[...] = jnp.max(no_ref[...], axis=1)


def _hinge_body(topk_ref, nmax_ref, out_ref):
    c = 1.0 - topk_ref[...]

    def chunk(i, acc):
        nm = nmax_ref[pl.ds(i * 512, 512), :]
        return acc + jnp.sum(jnp.maximum(nm + c, 0.0))

    acc = jax.lax.fori_loop(0, B // 512, chunk, jnp.float32(0.0))
    out_ref[0, 0] = acc / np.float32(B)


@jax.jit
def _run(ao, no, feats, proj):
    topk, nmax = pl.pallas_call(
        _main_body,
        grid=(B // BB,),
        in_specs=[
            pl.BlockSpec((BB, T), lambda i: (i, 0)),
            pl.BlockSpec((BB, T), lambda i: (i, 0)),
            pl.BlockSpec((BB, T, D), lambda i: (i, 0, 0)),
            pl.BlockSpec((D, K), lambda i: (0, 0)),
        ],
        out_specs=[
            pl.BlockSpec((BB,), lambda i: (i,)),
            pl.BlockSpec((BB,), lambda i: (i,)),
        ],
        out_shape=[
            jax.ShapeDtypeStruct((B,), jnp.float32),
            jax.ShapeDtypeStruct((B,), jnp.float32),
        ],
    )(ao, no, feats, proj)

    out = pl.pallas_call(
        _hinge_body,
        out_shape=jax.ShapeDtypeStruct((1, 1), jnp.float32),
        out_specs=pl.BlockSpec(memory_space=pltpu.SMEM),
    )(topk.reshape(1, B), nmax.reshape(B, 1))
    return out[0, 0]


def kernel(abnormal_outputs, normal_outputs, abnormal_features,
           normal_features, proj, sim_th, out_th):
    del normal_features, sim_th, out_th
    return _run(abnormal_outputs, normal_outputs, abnormal_features, proj)


# speedup vs baseline: 3.8303x; 3.8303x over previous
import functools

import jax
import jax.numpy as jnp
import numpy as np
from jax.experimental import pallas as pl
from jax.experimental.pallas import tpu as pltpu

B, T, D, K = 4096, 64, 256, 16
BB = 128

_Q_LO = 22
_Q_FRAC = np.float32(0.35 * (T - 1) - _Q_LO)
_NEG = np.float32(np.finfo(np.float32).min)


def _main_body(ao_ref, no_ref, f_ref, proj_ref, topk_ref, nmax_ref):
    f = f_ref[...].reshape(BB * T, D)
    scores = jax.lax.dot_general(
        f, proj_ref[...], (((1,), (0,)), ((), ())),
        preferred_element_type=jnp.float32)
    s3 = scores.reshape(BB, T, K)

    ao = ao_ref[...]
    le = (ao[:, :, None] <= ao[:, None, :]).astype(jnp.float32)
    cnt = jnp.sum(le, axis=1)
    v_lo = jnp.min(jnp.where(cnt >= _Q_LO + 1, ao, jnp.inf), axis=1)
    v_hi = jnp.min(jnp.where(cnt >= _Q_LO + 2, ao, jnp.inf), axis=1)
    th = v_lo + _Q_FRAC * (v_hi - v_lo)

    masked = jnp.where(ao >= th[:, None], ao, _NEG)
    m3 = masked[:, :, None]

    kio = jax.lax.broadcasted_iota(jnp.int32, (BB, T, K), 2)
    eq = s3 == jnp.max(s3, axis=2, keepdims=True)
    idx = jnp.min(jnp.where(eq, kio, K), axis=2, keepdims=True)
    val3 = jnp.where(kio == idx, m3, _NEG)

    cmax = jnp.max(val3, axis=1)
    present = cmax > _NEG
    vsum = jnp.sum(jnp.where(present, cmax, 0.0), axis=1)
    ncl = jnp.sum(present.astype(jnp.float32), axis=1)
    topk_ref[...] = vsum / jnp.maximum(ncl, 1.0)
    nmax_ref[...] = jnp.max(no_ref[...], axis=1)


def _hinge_body(topk_ref, nmax_ref, out_ref):
    c = 1.0 - topk_ref[...]

    def chunk(i, acc):
        nm = nmax_ref[pl.ds(i * 512, 512), :]
        return acc + jnp.sum(jnp.maximum(nm + c, 0.0))

    acc = jax.lax.fori_loop(0, B // 512, chunk, jnp.float32(0.0))
    out_ref[0, 0] = acc / np.float32(B)


@jax.jit
def _run(ao, no, feats, proj):
    topk, nmax = pl.pallas_call(
        _main_body,
        grid=(B // BB,),
        in_specs=[
            pl.BlockSpec((BB, T), lambda i: (i, 0)),
            pl.BlockSpec((BB, T), lambda i: (i, 0)),
            pl.BlockSpec((BB, T, D), lambda i: (i, 0, 0)),
            pl.BlockSpec((D, K), lambda i: (0, 0)),
        ],
        out_specs=[
            pl.BlockSpec((BB,), lambda i: (i,)),
            pl.BlockSpec((BB,), lambda i: (i,)),
        ],
        out_shape=[
            jax.ShapeDtypeStruct((B,), jnp.float32),
            jax.ShapeDtypeStruct((B,), jnp.float32),
        ],
    )(ao, no, feats, proj)

    out = pl.pallas_call(
        _hinge_body,
        out_shape=jax.ShapeDtypeStruct((1, 1), jnp.float32),
        out_specs=pl.BlockSpec(memory_space=pltpu.SMEM),
    )(topk.reshape(1, B), nmax.reshape(B, 1))
    return out[0, 0]


def kernel(abnormal_outputs, normal_outputs, abnormal_features,
           normal_features, proj, sim_th, out_th):
    del normal_features, sim_th, out_th
    return _run(abnormal_outputs, normal_outputs, abnormal_features, proj)


# speedup vs baseline: 5.7396x; 1.4985x over previous
import functools

import jax
import jax.numpy as jnp
import numpy as np
from jax.experimental import pallas as pl
from jax.experimental.pallas import tpu as pltpu

B, T, D, K = 4096, 64, 256, 16
BB = 128

_Q_LO = 22
_Q_FRAC = np.float32(0.35 * (T - 1) - _Q_LO)
_NEG = np.float32(np.finfo(np.float32).min)


def _main_body(ao_ref, no_ref, f_ref, proj_ref, topk_ref, nmax_ref):
    f = f_ref[...].reshape(BB * T, D)
    scores = jax.lax.dot_general(
        f, proj_ref[...], (((1,), (0,)), ((), ())),
        preferred_element_type=jnp.float32)
    s3 = scores.reshape(BB, T, K)

    ao = ao_ref[...]
    le = (ao[:, :, None] <= ao[:, None, :]).astype(jnp.float32)
    cnt = jnp.sum(le, axis=1)
    v_lo = jnp.min(jnp.where(cnt >= _Q_LO + 1, ao, jnp.inf), axis=1)
    v_hi = jnp.min(jnp.where(cnt >= _Q_LO + 2, ao, jnp.inf), axis=1)
    th = v_lo + _Q_FRAC * (v_hi - v_lo)

    masked = jnp.where(ao >= th[:, None], ao, _NEG)
    m3 = masked[:, :, None]

    kio = jax.lax.broadcasted_iota(jnp.int32, (BB, T, K), 2)
    idx = jnp.argmax(s3, axis=2, keepdims=True)
    val3 = jnp.where(kio == idx, m3, _NEG)

    cmax = jnp.max(val3, axis=1)
    present = cmax > _NEG
    vsum = jnp.sum(jnp.where(present, cmax, 0.0), axis=1)
    ncl = jnp.sum(present.astype(jnp.float32), axis=1)
    topk_ref[...] = vsum / jnp.maximum(ncl, 1.0)
    nmax_ref[...] = jnp.max(no_ref[...], axis=1)


def _hinge_body(topk_ref, nmax_ref, out_ref):
    c = 1.0 - topk_ref[...]

    def chunk(i, acc):
        nm = nmax_ref[pl.ds(i * 512, 512), :]
        return acc + jnp.sum(jnp.maximum(nm + c, 0.0))

    acc = jax.lax.fori_loop(0, B // 512, chunk, jnp.float32(0.0))
    out_ref[0, 0] = acc / np.float32(B)


@jax.jit
def _run(ao, no, feats, proj):
    topk, nmax = pl.pallas_call(
        _main_body,
        grid=(B // BB,),
        in_specs=[
            pl.BlockSpec((BB, T), lambda i: (i, 0)),
            pl.BlockSpec((BB, T), lambda i: (i, 0)),
            pl.BlockSpec((BB, T, D), lambda i: (i, 0, 0)),
            pl.BlockSpec((D, K), lambda i: (0, 0)),
        ],
        out_specs=[
            pl.BlockSpec((BB,), lambda i: (i,)),
            pl.BlockSpec((BB,), lambda i: (i,)),
        ],
        out_shape=[
            jax.ShapeDtypeStruct((B,), jnp.float32),
            jax.ShapeDtypeStruct((B,), jnp.float32),
        ],
    )(ao, no, feats, proj)

    out = pl.pallas_call(
        _hinge_body,
        out_shape=jax.ShapeDtypeStruct((1, 1), jnp.float32),
        out_specs=pl.BlockSpec(memory_space=pltpu.SMEM),
    )(topk.reshape(1, B), nmax.reshape(B, 1))
    return out[0, 0]


def kernel(abnormal_outputs, normal_outputs, abnormal_features,
           normal_features, proj, sim_th, out_th):
    del normal_features, sim_th, out_th
    return _run(abnormal_outputs, normal_outputs, abnormal_features, proj)


# speedup vs baseline: 5.8980x; 1.0276x over previous
import functools

import jax
import jax.numpy as jnp
import numpy as np
from jax.experimental import pallas as pl
from jax.experimental.pallas import tpu as pltpu

B, T, D, K = 4096, 64, 256, 16
BB = 256

_Q_LO = 22
_Q_FRAC = np.float32(0.35 * (T - 1) - _Q_LO)
_NEG = np.float32(np.finfo(np.float32).min)


def _main_body(ao_ref, no_ref, f_ref, proj_ref, topk_ref, nmax_ref):
    f = f_ref[...].reshape(BB * T, D)
    scores = jax.lax.dot_general(
        f, proj_ref[...], (((1,), (0,)), ((), ())),
        preferred_element_type=jnp.float32)
    s3 = scores.reshape(BB, T, K)

    ao = ao_ref[...]
    le = (ao[:, :, None] <= ao[:, None, :]).astype(jnp.float32)
    cnt = jnp.sum(le, axis=1)
    v_lo = jnp.min(jnp.where(cnt >= _Q_LO + 1, ao, jnp.inf), axis=1)
    v_hi = jnp.min(jnp.where(cnt >= _Q_LO + 2, ao, jnp.inf), axis=1)
    th = v_lo + _Q_FRAC * (v_hi - v_lo)

    masked = jnp.where(ao >= th[:, None], ao, _NEG)
    m3 = masked[:, :, None]

    kio = jax.lax.broadcasted_iota(jnp.int32, (BB, T, K), 2)
    idx = jnp.argmax(s3, axis=2, keepdims=True)
    val3 = jnp.where(kio == idx, m3, _NEG)

    cmax = jnp.max(val3, axis=1)
    present = cmax > _NEG
    vsum = jnp.sum(jnp.where(present, cmax, 0.0), axis=1)
    ncl = jnp.sum(present.astype(jnp.float32), axis=1)
    topk_ref[...] = vsum / jnp.maximum(ncl, 1.0)
    nmax_ref[...] = jnp.max(no_ref[...], axis=1)


def _hinge_body(topk_ref, nmax_ref, out_ref):
    c = 1.0 - topk_ref[...]

    def chunk(i, acc):
        nm = nmax_ref[pl.ds(i * 512, 512), :]
        return acc + jnp.sum(jnp.maximum(nm + c, 0.0))

    acc = jax.lax.fori_loop(0, B // 512, chunk, jnp.float32(0.0))
    out_ref[0, 0] = acc / np.float32(B)


@jax.jit
def _run(ao, no, feats, proj):
    topk, nmax = pl.pallas_call(
        _main_body,
        grid=(B // BB,),
        in_specs=[
            pl.BlockSpec((BB, T), lambda i: (i, 0)),
            pl.BlockSpec((BB, T), lambda i: (i, 0)),
            pl.BlockSpec((BB, T, D), lambda i: (i, 0, 0)),
            pl.BlockSpec((D, K), lambda i: (0, 0)),
        ],
        out_specs=[
            pl.BlockSpec((BB,), lambda i: (i,)),
            pl.BlockSpec((BB,), lambda i: (i,)),
        ],
        out_shape=[
            jax.ShapeDtypeStruct((B,), jnp.float32),
            jax.ShapeDtypeStruct((B,), jnp.float32),
        ],
    )(ao, no, feats, proj)

    out = pl.pallas_call(
        _hinge_body,
        out_shape=jax.ShapeDtypeStruct((1, 1), jnp.float32),
        out_specs=pl.BlockSpec(memory_space=pltpu.SMEM),
    )(topk.reshape(1, B), nmax.reshape(B, 1))
    return out[0, 0]


def kernel(abnormal_outputs, normal_outputs, abnormal_features,
           normal_features, proj, sim_th, out_th):
    del normal_features, sim_th, out_th
    return _run(abnormal_outputs, normal_outputs, abnormal_features, proj)


# speedup vs baseline: 5.9004x; 1.0004x over previous
import functools

import jax
import jax.numpy as jnp
import numpy as np
from jax.experimental import pallas as pl
from jax.experimental.pallas import tpu as pltpu

B, T, D, K = 4096, 64, 256, 16
BB = 256

_Q_LO = 22
_Q_FRAC = np.float32(0.35 * (T - 1) - _Q_LO)
_NEG = np.float32(np.finfo(np.float32).min)


def _main_body(ao_ref, no_ref, f_ref, proj_ref, topk_ref, nmax_ref):
    f = f_ref[...].reshape(BB * T, D)
    scores = jax.lax.dot_general(
        f, proj_ref[...], (((1,), (0,)), ((), ())),
        preferred_element_type=jnp.float32)
    s3 = scores.reshape(BB, T, K)

    ao = ao_ref[...]
    le = (ao[:, :, None] <= ao[:, None, :]).astype(jnp.float32)
    cnt = jnp.sum(le, axis=1)
    v_lo = jnp.min(jnp.where(cnt >= _Q_LO + 1, ao, jnp.inf), axis=1)
    v_hi = jnp.min(jnp.where(cnt >= _Q_LO + 2, ao, jnp.inf), axis=1)
    th = v_lo + _Q_FRAC * (v_hi - v_lo)

    masked = jnp.where(ao >= th[:, None], ao, _NEG)
    m3 = masked[:, :, None]

    kio = jax.lax.broadcasted_iota(jnp.int32, (BB, T, K), 2)
    idx = jnp.argmax(s3, axis=2, keepdims=True)
    val3 = jnp.where(kio == idx, m3, _NEG)

    cmax = jnp.max(val3, axis=1)
    present = cmax > _NEG
    vsum = jnp.sum(jnp.where(present, cmax, 0.0), axis=1)
    ncl = jnp.sum(present.astype(jnp.float32), axis=1)
    topk_ref[...] = vsum / jnp.maximum(ncl, 1.0)
    nmax_ref[...] = jnp.max(no_ref[...], axis=1)


def _hinge_body(topk_ref, nmax_ref, out_ref):
    c = 1.0 - topk_ref[...]

    def chunk(i, acc):
        nm = nmax_ref[pl.ds(i * 512, 512), :]
        return acc + jnp.sum(jnp.maximum(nm + c, 0.0))

    acc = jax.lax.fori_loop(0, B // 512, chunk, jnp.float32(0.0))
    out_ref[0, 0] = acc / np.float32(B)


@jax.jit
def _run(ao, no, feats, proj):
    topk, nmax = pl.pallas_call(
        _main_body,
        grid=(B // BB,),
        in_specs=[
            pl.BlockSpec((BB, T), lambda i: (i, 0)),
            pl.BlockSpec((BB, T), lambda i: (i, 0)),
            pl.BlockSpec((BB, T, D), lambda i: (i, 0, 0)),
            pl.BlockSpec((D, K), lambda i: (0, 0)),
        ],
        out_specs=[
            pl.BlockSpec((BB,), lambda i: (i,)),
            pl.BlockSpec((BB,), lambda i: (i,)),
        ],
        out_shape=[
            jax.ShapeDtypeStruct((B,), jnp.float32),
            jax.ShapeDtypeStruct((B,), jnp.float32),
        ],
        compiler_params=pltpu.CompilerParams(
            vmem_limit_bytes=120 * 1024 * 1024),
    )(ao, no, feats, proj)

    out = pl.pallas_call(
        _hinge_body,
        out_shape=jax.ShapeDtypeStruct((1, 1), jnp.float32),
        out_specs=pl.BlockSpec(memory_space=pltpu.SMEM),
    )(topk.reshape(1, B), nmax.reshape(B, 1))
    return out[0, 0]


def kernel(abnormal_outputs, normal_outputs, abnormal_features,
           normal_features, proj, sim_th, out_th):
    del normal_features, sim_th, out_th
    return _run(abnormal_outputs, normal_outputs, abnormal_features, proj)


# speedup vs baseline: 6.1146x; 1.0363x over previous
import functools

import jax
import jax.numpy as jnp
import numpy as np
from jax.experimental import pallas as pl
from jax.experimental.pallas import tpu as pltpu

B, T, D, K = 4096, 64, 256, 16
BB = 256
NSTEP = B // BB

_Q_LO = 22
_Q_FRAC = np.float32(0.35 * (T - 1) - _Q_LO)
_NEG = np.float32(np.finfo(np.float32).min)


def _body(ao_ref, no_ref, f_ref, proj_ref, out_ref, tk_s, nm_s):
    i = pl.program_id(0)

    f = f_ref[...].reshape(BB * T, D)
    scores = jax.lax.dot_general(
        f, proj_ref[...], (((1,), (0,)), ((), ())),
        preferred_element_type=jnp.float32)
    s3 = scores.reshape(BB, T, K)

    ao = ao_ref[...]
    le = (ao[:, :, None] <= ao[:, None, :]).astype(jnp.float32)
    cnt = jnp.sum(le, axis=1)
    v_lo = jnp.min(jnp.where(cnt >= _Q_LO + 1, ao, jnp.inf), axis=1)
    v_hi = jnp.min(jnp.where(cnt >= _Q_LO + 2, ao, jnp.inf), axis=1)
    th = v_lo + _Q_FRAC * (v_hi - v_lo)

    masked = jnp.where(ao >= th[:, None], ao, _NEG)
    m3 = masked[:, :, None]

    kio = jax.lax.broadcasted_iota(jnp.int32, (BB, T, K), 2)
    idx = jnp.argmax(s3, axis=2, keepdims=True)
    val3 = jnp.where(kio == idx, m3, _NEG)

    cmax = jnp.max(val3, axis=1)
    present = cmax > _NEG
    vsum = jnp.sum(jnp.where(present, cmax, 0.0), axis=1)
    ncl = jnp.sum(present.astype(jnp.float32), axis=1)
    tk_s[0, pl.ds(i * BB, BB)] = vsum / jnp.maximum(ncl, 1.0)
    nm_s[pl.ds(i * BB, BB), :] = jnp.max(no_ref[...], axis=1)[:, None]

    @pl.when(i == NSTEP - 1)
    def _hinge():
        c = 1.0 - tk_s[...]

        def chunk(j, acc):
            nmj = nm_s[pl.ds(j * 512, 512), :]
            return acc + jnp.sum(jnp.maximum(nmj + c, 0.0))

        acc = jax.lax.fori_loop(0, B // 512, chunk, jnp.float32(0.0))
        out_ref[0, 0] = acc / np.float32(B)


@jax.jit
def _run(ao, no, feats, proj):
    out = pl.pallas_call(
        _body,
        grid=(NSTEP,),
        in_specs=[
            pl.BlockSpec((BB, T), lambda i: (i, 0)),
            pl.BlockSpec((BB, T), lambda i: (i, 0)),
            pl.BlockSpec((BB, T, D), lambda i: (i, 0, 0)),
            pl.BlockSpec((D, K), lambda i: (0, 0)),
        ],
        out_specs=pl.BlockSpec(memory_space=pltpu.SMEM),
        out_shape=jax.ShapeDtypeStruct((1, 1), jnp.float32),
        scratch_shapes=[
            pltpu.VMEM((1, B), jnp.float32),
            pltpu.VMEM((B, 1), jnp.float32),
        ],
        compiler_params=pltpu.CompilerParams(
            vmem_limit_bytes=120 * 1024 * 1024),
    )(ao, no, feats, proj)
    return out[0, 0]


def kernel(abnormal_outputs, normal_outputs, abnormal_features,
           normal_features, proj, sim_th, out_th):
    del normal_features, sim_th, out_th
    return _run(abnormal_outputs, normal_outputs, abnormal_features, proj)
